# baseline (speedup 1.0000x reference)
import jax
import jax.numpy as jnp
from jax import lax
from jax.experimental import pallas as pl
from jax.experimental.pallas import tpu as pltpu
from jax.experimental.pallas import tpu_sc as plsc

N = 10000
E = 320000
D = 128

NC = 2
NS = 16
NW = NC * NS

G = 128
NG_TILE = 81
E_PAD = NW * NG_TILE * G
RSLOTS = 3

ROWS_A = 632
ROWS_B = N - (NS - 1) * ROWS_A


def _linear_body(x_ref, wt_ref, b_ref, o_ref):
    o_ref[...] = (
        jnp.dot(x_ref[...], wt_ref[...], preferred_element_type=jnp.float32)
        + b_ref[...]
    )


def _linear(x, wt, b2d):
    blk = 2000
    return pl.pallas_call(
        _linear_body,
        grid=(N // blk,),
        in_specs=[
            pl.BlockSpec((blk, D), lambda i: (i, 0)),
            pl.BlockSpec((D, D), lambda i: (0, 0)),
            pl.BlockSpec((1, D), lambda i: (0, 0)),
        ],
        out_specs=pl.BlockSpec((blk, D), lambda i: (i, 0)),
        out_shape=jax.ShapeDtypeStruct((N, D), jnp.float32),
    )(x, wt, b2d)


def _combine_body(p_ref, o_ref):
    o_ref[...] = jnp.maximum(p_ref[0] + p_ref[1], 0.0)


def _combine(partials):
    blk = 2000
    return pl.pallas_call(
        _combine_body,
        grid=(N // blk,),
        in_specs=[pl.BlockSpec((NC, blk, D), lambda i: (0, i, 0))],
        out_specs=pl.BlockSpec((blk, D), lambda i: (i, 0)),
        out_shape=jax.ShapeDtypeStruct((N, D), jnp.float32),
    )(partials)


def _spmm_body(h_hbm, src_hbm, dst_hbm, w_hbm, zeros_hbm, out_hbm,
               srcb, dstb, wb, rows, acc, sem_st, sem_g, sem_s):
    c = lax.axis_index("c")
    s = lax.axis_index("s")
    wid = c * NS + s
    g0 = wid * NG_TILE

    row0 = s * ROWS_A

    @pl.when(s < NS - 1)
    def _():
        pltpu.sync_copy(zeros_hbm.at[pl.ds(row0, ROWS_A)],
                        acc.at[pl.ds(row0, ROWS_A)])

    @pl.when(s == NS - 1)
    def _():
        pltpu.sync_copy(zeros_hbm.at[pl.ds(row0, ROWS_B)],
                        acc.at[pl.ds(row0, ROWS_B)])

    plsc.subcore_barrier()

    def stage(g, r):
        base = (g0 + g) * G
        pltpu.async_copy(src_hbm.at[pl.ds(base, G)], srcb[r], sem_st[r])
        pltpu.async_copy(dst_hbm.at[pl.ds(base, G)], dstb[r], sem_st[r])
        pltpu.async_copy(w_hbm.at[pl.ds(base, G)], wb[r], sem_st[r])

    def wait_stage(g, r):
        base = (g0 + g) * G
        pltpu.make_async_copy(src_hbm.at[pl.ds(base, G)], srcb[r],
                              sem_st[r]).wait()
        pltpu.make_async_copy(dst_hbm.at[pl.ds(base, G)], dstb[r],
                              sem_st[r]).wait()
        pltpu.make_async_copy(w_hbm.at[pl.ds(base, G)], wb[r],
                              sem_st[r]).wait()

    def gather(r):
        pltpu.async_copy(h_hbm.at[srcb[r]], rows[r], sem_g[r])

    def wait_gather(r):
        pltpu.make_async_copy(h_hbm.at[srcb[r]], rows[r], sem_g[r]).wait()

    def scatter(r):
        pltpu.async_copy(rows[r], acc.at[dstb[r]], sem_s[r], add=True)

    def wait_scatter(r):
        pltpu.make_async_copy(rows[r], acc.at[dstb[r]], sem_s[r]).wait()

    def multiply(r):
        rows_r = rows[r]
        wb_r = wb[r]

        @pl.loop(0, G // 16)
        def _edge16(blk16):
            wv16 = wb_r[pl.ds(blk16 * 16, 16)]
            for i in range(16):
                w = wv16[i]
                e = blk16 * 16 + i
                for jj in range(D // 16):
                    sl = pl.ds(jj * 16, 16)
                    rows_r[e, sl] = rows_r[e, sl] * w

    stage(0, 0)
    stage(1, 1)
    wait_stage(0, 0)
    gather(0)

    def sub_body(g, r):
        rn = (r + 1) % RSLOTS

        @pl.when(g + 2 < NG_TILE)
        def _():
            stage(g + 2, (r + 2) % RSLOTS)

        @pl.when(g >= 2)
        def _():
            wait_scatter(rn)

        @pl.when(g + 1 < NG_TILE)
        def _():
            wait_stage(g + 1, rn)
            gather(rn)

        wait_gather(r)
        multiply(r)
        scatter(r)

    @pl.loop(0, NG_TILE // RSLOTS)
    def _ring(m):
        g = m * RSLOTS
        sub_body(g, 0)
        sub_body(g + 1, 1)
        sub_body(g + 2, 2)

    wait_scatter((NG_TILE - 2) % RSLOTS)
    wait_scatter((NG_TILE - 1) % RSLOTS)

    plsc.subcore_barrier()

    @pl.when(s < NS - 1)
    def _():
        pltpu.sync_copy(acc.at[pl.ds(row0, ROWS_A)],
                        out_hbm.at[c, pl.ds(row0, ROWS_A)])

    @pl.when(s == NS - 1)
    def _():
        pltpu.sync_copy(acc.at[pl.ds(row0, ROWS_B)],
                        out_hbm.at[c, pl.ds(row0, ROWS_B)])


def _spmm(h, src, dst, w, zeros):
    mesh = plsc.VectorSubcoreMesh(core_axis_name="c", subcore_axis_name="s")
    kern = pl.kernel(
        _spmm_body,
        out_type=jax.ShapeDtypeStruct((NC, N, D), jnp.float32),
        mesh=mesh,
        scratch_types=[
            [pltpu.VMEM((G,), jnp.int32) for _ in range(RSLOTS)],
            [pltpu.VMEM((G,), jnp.int32) for _ in range(RSLOTS)],
            [pltpu.VMEM((G,), jnp.float32) for _ in range(RSLOTS)],
            [pltpu.VMEM((G, D), jnp.float32) for _ in range(RSLOTS)],
            pltpu.VMEM_SHARED((N, D), jnp.float32),
            [pltpu.SemaphoreType.DMA for _ in range(RSLOTS)],
            [pltpu.SemaphoreType.DMA for _ in range(RSLOTS)],
            [pltpu.SemaphoreType.DMA for _ in range(RSLOTS)],
        ],
    )
    return kern(h, src, dst, w, zeros)


def kernel(x, edge_index, edge_weight, W, b):
    wt = W.T
    b2d = b.reshape(1, D)
    h = _linear(x, wt, b2d)

    pad = E_PAD - E
    src = jnp.concatenate([edge_index[1], jnp.zeros((pad,), jnp.int32)])
    dst = jnp.concatenate([edge_index[0], jnp.zeros((pad,), jnp.int32)])
    w_pad = jnp.concatenate([edge_weight, jnp.zeros((pad,), jnp.float32)])

    zeros = jnp.zeros((N, D), dtype=jnp.float32)
    partials = _spmm(h, src, dst, w_pad, zeros)
    return _combine(partials)


# speedup vs baseline: 1.0416x; 1.0416x over previous
import jax
import jax.numpy as jnp
from jax import lax
from jax.experimental import pallas as pl
from jax.experimental.pallas import tpu as pltpu
from jax.experimental.pallas import tpu_sc as plsc

N = 10000
E = 320000
D = 128

NC = 2
NS = 16
NW = NC * NS

G = 128
NG = 2560
NG_TILE = NG // NW
E_ALLOC = (NG + NW) * G

ROWS_A = 632
ROWS_B = N - (NS - 1) * ROWS_A


def _linear_body(x_ref, wt_ref, b_ref, o_ref):
    o_ref[...] = (
        jnp.dot(x_ref[...], wt_ref[...], preferred_element_type=jnp.float32)
        + b_ref[...]
    )


def _linear(x, wt, b2d):
    blk = 2000
    return pl.pallas_call(
        _linear_body,
        grid=(N // blk,),
        in_specs=[
            pl.BlockSpec((blk, D), lambda i: (i, 0)),
            pl.BlockSpec((D, D), lambda i: (0, 0)),
            pl.BlockSpec((1, D), lambda i: (0, 0)),
        ],
        out_specs=pl.BlockSpec((blk, D), lambda i: (i, 0)),
        out_shape=jax.ShapeDtypeStruct((N, D), jnp.float32),
    )(x, wt, b2d)


def _combine_body(p_ref, o_ref):
    o_ref[...] = jnp.maximum(p_ref[0] + p_ref[1], 0.0)


def _combine(partials):
    blk = 2000
    return pl.pallas_call(
        _combine_body,
        grid=(N // blk,),
        in_specs=[pl.BlockSpec((NC, blk, D), lambda i: (0, i, 0))],
        out_specs=pl.BlockSpec((blk, D), lambda i: (i, 0)),
        out_shape=jax.ShapeDtypeStruct((N, D), jnp.float32),
    )(partials)


def _spmm_body(h_hbm, src_hbm, dst_hbm, w_hbm, zeros_hbm, out_hbm,
               srcb, dstb, wb, rows, acc, sem_g):
    c = lax.axis_index("c")
    s = lax.axis_index("s")
    wid = c * NS + s

    row0 = s * ROWS_A

    @pl.when(s < NS - 1)
    def _():
        pltpu.sync_copy(zeros_hbm.at[pl.ds(row0, ROWS_A)],
                        acc.at[pl.ds(row0, ROWS_A)])

    @pl.when(s == NS - 1)
    def _():
        pltpu.sync_copy(zeros_hbm.at[pl.ds(row0, ROWS_B)],
                        acc.at[pl.ds(row0, ROWS_B)])

    plsc.subcore_barrier()

    def stage(j, r):
        base = j * G
        pltpu.sync_copy(src_hbm.at[pl.ds(base, G)], srcb[r])
        pltpu.sync_copy(dst_hbm.at[pl.ds(base, G)], dstb[r])
        pltpu.sync_copy(w_hbm.at[pl.ds(base, G)], wb[r])

    def gather(r):
        pltpu.async_copy(h_hbm.at[srcb[r]], rows[r], sem_g[r])

    def wait_gather(r):
        pltpu.make_async_copy(h_hbm.at[srcb[r]], rows[r], sem_g[r]).wait()

    def multiply(r):
        rows_r = rows[r]
        wb_r = wb[r]

        @pl.loop(0, G // 16)
        def _edge16(blk16):
            wv16 = wb_r[pl.ds(blk16 * 16, 16)]
            for i in range(16):
                w = wv16[i]
                e = blk16 * 16 + i
                for jj in range(D // 16):
                    sl = pl.ds(jj * 16, 16)
                    rows_r[e, sl] = rows_r[e, sl] * w

    def scatter(r):
        pltpu.sync_copy(rows[r], acc.at[dstb[r]], add=True)

    stage(wid, 0)
    gather(0)

    @pl.loop(0, NG_TILE // 2)
    def _pair(m):
        ja = wid + NW * 2 * m
        jb = ja + NW

        stage(jb, 1)
        gather(1)
        wait_gather(0)
        multiply(0)
        scatter(0)

        stage(ja + 2 * NW, 0)
        gather(0)
        wait_gather(1)
        multiply(1)
        scatter(1)

    wait_gather(0)

    plsc.subcore_barrier()

    @pl.when(s < NS - 1)
    def _():
        pltpu.sync_copy(acc.at[pl.ds(row0, ROWS_A)],
                        out_hbm.at[c, pl.ds(row0, ROWS_A)])

    @pl.when(s == NS - 1)
    def _():
        pltpu.sync_copy(acc.at[pl.ds(row0, ROWS_B)],
                        out_hbm.at[c, pl.ds(row0, ROWS_B)])


def _spmm(h, src, dst, w, zeros):
    mesh = plsc.VectorSubcoreMesh(core_axis_name="c", subcore_axis_name="s")
    kern = pl.kernel(
        _spmm_body,
        out_type=jax.ShapeDtypeStruct((NC, N, D), jnp.float32),
        mesh=mesh,
        scratch_types=[
            [pltpu.VMEM((G,), jnp.int32) for _ in range(2)],
            [pltpu.VMEM((G,), jnp.int32) for _ in range(2)],
            [pltpu.VMEM((G,), jnp.float32) for _ in range(2)],
            [pltpu.VMEM((G, D), jnp.float32) for _ in range(2)],
            pltpu.VMEM_SHARED((N, D), jnp.float32),
            [pltpu.SemaphoreType.DMA for _ in range(2)],
        ],
    )
    return kern(h, src, dst, w, zeros)


def kernel(x, edge_index, edge_weight, W, b):
    wt = W.T
    b2d = b.reshape(1, D)
    h = _linear(x, wt, b2d)

    pad = E_ALLOC - E
    src = jnp.concatenate([edge_index[1], jnp.zeros((pad,), jnp.int32)])
    dst = jnp.concatenate([edge_index[0], jnp.zeros((pad,), jnp.int32)])
    w_pad = jnp.concatenate([edge_weight, jnp.zeros((pad,), jnp.float32)])

    zeros = jnp.zeros((N, D), dtype=jnp.float32)
    partials = _spmm(h, src, dst, w_pad, zeros)
    return _combine(partials)


# speedup vs baseline: 1.3513x; 1.2973x over previous
import jax
import jax.numpy as jnp
from jax import lax
from jax.experimental import pallas as pl
from jax.experimental.pallas import tpu as pltpu
from jax.experimental.pallas import tpu_sc as plsc

N = 10000
E = 320000
D = 128

NC = 2
NS = 16
NW = NC * NS

G = 128
NG = 2560
NG_TILE = NG // NW
E_ALLOC = (NG + NW) * G

ROWS_A = 632
ROWS_B = N - (NS - 1) * ROWS_A


def _linear_body(x_ref, wt_ref, b_ref, o_ref):
    o_ref[...] = (
        jnp.dot(x_ref[...], wt_ref[...], preferred_element_type=jnp.float32)
        + b_ref[...]
    )


def _linear(x, wt, b2d):
    blk = 2000
    return pl.pallas_call(
        _linear_body,
        grid=(N // blk,),
        in_specs=[
            pl.BlockSpec((blk, D), lambda i: (i, 0)),
            pl.BlockSpec((D, D), lambda i: (0, 0)),
            pl.BlockSpec((1, D), lambda i: (0, 0)),
        ],
        out_specs=pl.BlockSpec((blk, D), lambda i: (i, 0)),
        out_shape=jax.ShapeDtypeStruct((N, D), jnp.float32),
    )(x, wt, b2d)


def _combine_body(p_ref, o_ref):
    o_ref[...] = jnp.maximum(p_ref[0] + p_ref[1], 0.0)


def _combine(partials):
    blk = 2000
    return pl.pallas_call(
        _combine_body,
        grid=(N // blk,),
        in_specs=[pl.BlockSpec((NC, blk, D), lambda i: (0, i, 0))],
        out_specs=pl.BlockSpec((blk, D), lambda i: (i, 0)),
        out_shape=jax.ShapeDtypeStruct((N, D), jnp.float32),
    )(partials)


def _spmm_body(h_hbm, src_hbm, dst_hbm, w_hbm, zeros_hbm, out_hbm,
               srcb, dstb, wb, rows, acc, sem_st, sem_g, sem_s):
    c = lax.axis_index("c")
    s = lax.axis_index("s")
    wid = c * NS + s

    row0 = s * ROWS_A

    @pl.when(s < NS - 1)
    def _():
        pltpu.sync_copy(zeros_hbm.at[pl.ds(row0, ROWS_A)],
                        acc.at[pl.ds(row0, ROWS_A)])

    @pl.when(s == NS - 1)
    def _():
        pltpu.sync_copy(zeros_hbm.at[pl.ds(row0, ROWS_B)],
                        acc.at[pl.ds(row0, ROWS_B)])

    plsc.subcore_barrier()

    def stage(j, r):
        base = j * G
        pltpu.async_copy(src_hbm.at[pl.ds(base, G)], srcb[r], sem_st[r])
        pltpu.async_copy(dst_hbm.at[pl.ds(base, G)], dstb[r], sem_st[r])
        pltpu.async_copy(w_hbm.at[pl.ds(base, G)], wb[r], sem_st[r])

    def wait_stage(j, r):
        base = j * G
        pltpu.make_async_copy(src_hbm.at[pl.ds(base, G)], srcb[r],
                              sem_st[r]).wait()
        pltpu.make_async_copy(dst_hbm.at[pl.ds(base, G)], dstb[r],
                              sem_st[r]).wait()
        pltpu.make_async_copy(w_hbm.at[pl.ds(base, G)], wb[r],
                              sem_st[r]).wait()

    def gather(r):
        pltpu.async_copy(h_hbm.at[srcb[r]], rows[r], sem_g[r])

    def wait_gather(r):
        pltpu.make_async_copy(h_hbm.at[srcb[r]], rows[r], sem_g[r]).wait()

    def multiply(r):
        rows_r = rows[r]
        wb_r = wb[r]

        @pl.loop(0, G // 16)
        def _edge16(blk16):
            wv16 = wb_r[pl.ds(blk16 * 16, 16)]
            for i in range(16):
                w = wv16[i]
                e = blk16 * 16 + i
                for jj in range(D // 16):
                    sl = pl.ds(jj * 16, 16)
                    rows_r[e, sl] = rows_r[e, sl] * w

    def scatter(r):
        pltpu.async_copy(rows[r], acc.at[dstb[r]], sem_s[r], add=True)

    def wait_scatter(r):
        pltpu.make_async_copy(rows[r], acc.at[dstb[r]], sem_s[r]).wait()

    @pl.loop(0, NG_TILE // 2)
    def _pair(m):
        ja = wid + NW * 2 * m
        jb = ja + NW

        stage(ja, 0)
        stage(jb, 1)
        wait_stage(ja, 0)
        wait_stage(jb, 1)
        gather(0)
        gather(1)
        wait_gather(0)
        wait_gather(1)
        multiply(0)
        multiply(1)
        scatter(0)
        scatter(1)
        wait_scatter(0)
        wait_scatter(1)

    plsc.subcore_barrier()

    @pl.when(s < NS - 1)
    def _():
        pltpu.sync_copy(acc.at[pl.ds(row0, ROWS_A)],
                        out_hbm.at[c, pl.ds(row0, ROWS_A)])

    @pl.when(s == NS - 1)
    def _():
        pltpu.sync_copy(acc.at[pl.ds(row0, ROWS_B)],
                        out_hbm.at[c, pl.ds(row0, ROWS_B)])


def _spmm(h, src, dst, w, zeros):
    mesh = plsc.VectorSubcoreMesh(core_axis_name="c", subcore_axis_name="s")
    kern = pl.kernel(
        _spmm_body,
        out_type=jax.ShapeDtypeStruct((NC, N, D), jnp.float32),
        mesh=mesh,
        scratch_types=[
            [pltpu.VMEM((G,), jnp.int32) for _ in range(2)],
            [pltpu.VMEM((G,), jnp.int32) for _ in range(2)],
            [pltpu.VMEM((G,), jnp.float32) for _ in range(2)],
            [pltpu.VMEM((G, D), jnp.float32) for _ in range(2)],
            pltpu.VMEM_SHARED((N, D), jnp.float32),
            [pltpu.SemaphoreType.DMA for _ in range(2)],
            [pltpu.SemaphoreType.DMA for _ in range(2)],
            [pltpu.SemaphoreType.DMA for _ in range(2)],
        ],
    )
    return kern(h, src, dst, w, zeros)


def kernel(x, edge_index, edge_weight, W, b):
    wt = W.T
    b2d = b.reshape(1, D)
    h = _linear(x, wt, b2d)

    pad = E_ALLOC - E
    src = jnp.concatenate([edge_index[1], jnp.zeros((pad,), jnp.int32)])
    dst = jnp.concatenate([edge_index[0], jnp.zeros((pad,), jnp.int32)])
    w_pad = jnp.concatenate([edge_weight, jnp.zeros((pad,), jnp.float32)])

    zeros = jnp.zeros((N, D), dtype=jnp.float32)
    partials = _spmm(h, src, dst, w_pad, zeros)
    return _combine(partials)


# speedup vs baseline: 1.9582x; 1.4491x over previous
import functools

import jax
import jax.numpy as jnp
from jax import lax
from jax.experimental import pallas as pl
from jax.experimental.pallas import tpu as pltpu
from jax.experimental.pallas import tpu_sc as plsc

N = 10000
E = 320000
D = 128

NC = 2
NS = 16
G = 128
NGROUPS = E // G
ROWS_A = 632
ROWS_B = N - (NS - 1) * ROWS_A


def _linear_body(x_ref, wt_ref, b_ref, o_ref):
    o_ref[...] = (
        jnp.dot(x_ref[...], wt_ref[...], preferred_element_type=jnp.float32)
        + b_ref[...]
    )


def _linear(x, wt, b2d):
    blk = 2000
    return pl.pallas_call(
        _linear_body,
        grid=(N // blk,),
        in_specs=[
            pl.BlockSpec((blk, D), lambda i: (i, 0)),
            pl.BlockSpec((D, D), lambda i: (0, 0)),
            pl.BlockSpec((1, D), lambda i: (0, 0)),
        ],
        out_specs=pl.BlockSpec((blk, D), lambda i: (i, 0)),
        out_shape=jax.ShapeDtypeStruct((N, D), jnp.float32),
    )(x, wt, b2d)


def _combine_body(p_ref, o_ref):
    o_ref[...] = jnp.maximum(p_ref[0] + p_ref[1], 0.0)


def _combine(partials):
    blk = 2000
    return pl.pallas_call(
        _combine_body,
        grid=(N // blk,),
        in_specs=[pl.BlockSpec((NC, blk, D), lambda i: (0, i, 0))],
        out_specs=pl.BlockSpec((blk, D), lambda i: (i, 0)),
        out_shape=jax.ShapeDtypeStruct((N, D), jnp.float32),
    )(partials)


def _spmm_body(h_hbm, src_hbm, dst_hbm, w_hbm, zeros_hbm, out_hbm,
               idx_src, idx_dst, wts, rows, acc, sem):
    c = lax.axis_index("c")
    s = lax.axis_index("s")
    wid = c * NS + s

    row0 = s * ROWS_A

    @pl.when(s < NS - 1)
    def _():
        pltpu.sync_copy(zeros_hbm.at[pl.ds(row0, ROWS_A)],
                        acc.at[pl.ds(row0, ROWS_A)])

    @pl.when(s == NS - 1)
    def _():
        pltpu.sync_copy(zeros_hbm.at[pl.ds(row0, ROWS_B)],
                        acc.at[pl.ds(row0, ROWS_B)])

    plsc.subcore_barrier()

    @pl.loop(wid, NGROUPS, step=NC * NS)
    def _group(j):
        base = j * G
        pltpu.sync_copy(src_hbm.at[pl.ds(base, G)], idx_src)
        pltpu.sync_copy(dst_hbm.at[pl.ds(base, G)], idx_dst)
        pltpu.sync_copy(w_hbm.at[pl.ds(base, G)], wts)
        pltpu.async_copy(h_hbm.at[idx_src], rows, sem).wait()

        @pl.loop(0, G // 16)
        def _edge16(g):
            wv16 = wts[pl.ds(g * 16, 16)]
            for i in range(16):
                w = wv16[i]
                e = g * 16 + i
                for jj in range(D // 16):
                    sl = pl.ds(jj * 16, 16)
                    rows[e, sl] = rows[e, sl] * w

        pltpu.sync_copy(rows, acc.at[idx_dst], add=True)

    plsc.subcore_barrier()

    @pl.when(s < NS - 1)
    def _():
        pltpu.sync_copy(acc.at[pl.ds(row0, ROWS_A)],
                        out_hbm.at[c, pl.ds(row0, ROWS_A)])

    @pl.when(s == NS - 1)
    def _():
        pltpu.sync_copy(acc.at[pl.ds(row0, ROWS_B)],
                        out_hbm.at[c, pl.ds(row0, ROWS_B)])


def _spmm(h, src, dst, w, zeros):
    mesh = plsc.VectorSubcoreMesh(core_axis_name="c", subcore_axis_name="s")
    kern = pl.kernel(
        _spmm_body,
        out_type=jax.ShapeDtypeStruct((NC, N, D), jnp.float32),
        mesh=mesh,
        scratch_types=[
            pltpu.VMEM((G,), jnp.int32),
            pltpu.VMEM((G,), jnp.int32),
            pltpu.VMEM((G,), jnp.float32),
            pltpu.VMEM((G, D), jnp.float32),
            pltpu.VMEM_SHARED((N, D), jnp.float32),
            pltpu.SemaphoreType.DMA,
        ],
    )
    return kern(h, src, dst, w, zeros)


def kernel(x, edge_index, edge_weight, W, b):
    wt = W.T
    b2d = b.reshape(1, D)
    h = _linear(x, wt, b2d)
    src = edge_index[1]
    dst = edge_index[0]
    zeros = jnp.zeros((N, D), dtype=jnp.float32)
    partials = _spmm(h, src, dst, edge_weight, zeros)
    return _combine(partials)
